# baseline (speedup 1.0000x reference)
import jax
import jax.numpy as jnp
from jax import lax
from jax.experimental import pallas as pl
from jax.experimental.pallas import tpu as pltpu
from jax.experimental.pallas import tpu_sc as plsc

NUM_CLASS = 100000
BATCH = 4096
CTX_DIM = 512
N_CLS_CTX = 4
PROMPT_LEN = 4
SEQ = 77
SUFFIX_LEN = SEQ - (2 * PROMPT_LEN + 1 + N_CLS_CTX)

NC, NS = 2, 16
NW = NC * NS
BPW = BATCH // NW
CHUNK = 16
NCHUNK = BPW // CHUNK

K = 64
NSTEP = BATCH // K
NSPLIT = 4
HEAD = 5
ROW_CLS = 5
ROW_MID = 9
ROW_SFX = 13


def _sc_gather_body(label_hbm, cls_hbm, out_hbm, idx_v, buf0, buf1, gsem, ssem):
    wid = lax.axis_index("s") * NC + lax.axis_index("c")
    base = wid * BPW
    pltpu.sync_copy(label_hbm.at[pl.ds(base, BPW)], idx_v)

    bufs = [buf0, buf1]
    g_h = [None] * NCHUNK
    s_h = [None] * NCHUNK

    def gather(c):
        return pltpu.async_copy(
            cls_hbm.at[idx_v.at[pl.ds(c * CHUNK, CHUNK)]], bufs[c % 2], gsem)

    g_h[0] = gather(0)
    for c in range(NCHUNK):
        if c + 1 < NCHUNK:
            if c - 1 >= 0:
                s_h[c - 1].wait()
            g_h[c + 1] = gather(c + 1)
        g_h[c].wait()
        s_h[c] = pltpu.async_copy(
            bufs[c % 2], out_hbm.at[pl.ds(base + c * CHUNK, CHUNK)], ssem)
    s_h[NCHUNK - 2].wait()
    s_h[NCHUNK - 1].wait()


def _sc_gather(label, cls_ctx):
    mesh = plsc.VectorSubcoreMesh(
        core_axis_name="c", subcore_axis_name="s",
        num_cores=NC, num_subcores=NS)
    return pl.kernel(
        _sc_gather_body,
        out_type=jax.ShapeDtypeStruct((BATCH, N_CLS_CTX, CTX_DIM),
                                      jnp.float32),
        mesh=mesh,
        scratch_types=[
            pltpu.VMEM((BPW,), jnp.int32),
            pltpu.VMEM((CHUNK, N_CLS_CTX, CTX_DIM), jnp.float32),
            pltpu.VMEM((CHUNK, N_CLS_CTX, CTX_DIM), jnp.float32),
            pltpu.SemaphoreType.DMA,
            pltpu.SemaphoreType.DMA,
        ],
        name="cls_gather_sc",
    )(label, cls_ctx)


def _tc_fill_body(head_ref, mid_ref, sfx_ref, clsg_any, out_any,
                  rep, clsbuf, lsem, wsem):
    i = pl.program_id(0)

    KP = K // NSPLIT

    def store_descs(step):
        return [
            pltpu.make_async_copy(
                rep.at[step % 2, pl.ds(p * KP, KP)],
                out_any.at[pl.ds(step * K + p * KP, KP)], wsem)
            for p in range(NSPLIT)
        ]

    def load_desc(step):
        return pltpu.make_async_copy(
            clsg_any.at[pl.ds(step * K, K)], clsbuf.at[step % 2], lsem)

    @pl.when(i == 0)
    def _init():
        for j in range(2):
            rep[j, :, 0:HEAD, :] = jnp.broadcast_to(
                head_ref[...][None], (K, HEAD, CTX_DIM))
            rep[j, :, ROW_MID:ROW_SFX, :] = jnp.broadcast_to(
                mid_ref[...][None], (K, PROMPT_LEN, CTX_DIM))
            rep[j, :, ROW_SFX:SEQ, :] = jnp.broadcast_to(
                sfx_ref[...][None], (K, SUFFIX_LEN, CTX_DIM))
        load_desc(0).start()

    @pl.when(i >= 2)
    def _drain_prev():
        for d in store_descs(i - 2):
            d.wait()

    @pl.when(i + 1 < NSTEP)
    def _prefetch():
        load_desc(i + 1).start()

    load_desc(i).wait()

    @pl.when(i % 2 == 0)
    def _ins0():
        rep[0, :, ROW_CLS:ROW_MID, :] = clsbuf[0]

    @pl.when(i % 2 == 1)
    def _ins1():
        rep[1, :, ROW_CLS:ROW_MID, :] = clsbuf[1]

    for d in store_descs(i):
        d.start()

    @pl.when(i == NSTEP - 1)
    def _drain_last():
        for d in store_descs(i - 1):
            d.wait()
        for d in store_descs(i):
            d.wait()


def _tc_fill(head_c, mid_c, sfx_c, cls_g):
    return pl.pallas_call(
        _tc_fill_body,
        grid=(NSTEP,),
        in_specs=[
            pl.BlockSpec((HEAD, CTX_DIM), lambda i: (0, 0)),
            pl.BlockSpec((PROMPT_LEN, CTX_DIM), lambda i: (0, 0)),
            pl.BlockSpec((SUFFIX_LEN, CTX_DIM), lambda i: (0, 0)),
            pl.BlockSpec(memory_space=pl.ANY),
        ],
        out_specs=pl.BlockSpec(memory_space=pl.ANY),
        out_shape=jax.ShapeDtypeStruct((BATCH, SEQ, CTX_DIM), jnp.float32),
        scratch_shapes=[
            pltpu.VMEM((2, K, SEQ, CTX_DIM), jnp.float32),
            pltpu.VMEM((2, K, N_CLS_CTX, CTX_DIM), jnp.float32),
            pltpu.SemaphoreType.DMA,
            pltpu.SemaphoreType.DMA,
        ],
        compiler_params=pltpu.CompilerParams(
            dimension_semantics=("arbitrary",)),
        name="prompt_fill_tc",
    )(head_c, mid_c, sfx_c, cls_g)


def kernel(label, cls_ctx, token_prefix, token_suffix, prefix_prompt,
           suffix_prompt):
    cls_g = _sc_gather(label, cls_ctx)
    head_c = jnp.concatenate(
        [token_prefix.reshape(1, CTX_DIM),
         prefix_prompt.reshape(PROMPT_LEN, CTX_DIM)], axis=0)
    mid_c = suffix_prompt.reshape(PROMPT_LEN, CTX_DIM)
    sfx_c = token_suffix.reshape(SUFFIX_LEN, CTX_DIM)
    return _tc_fill(head_c, mid_c, sfx_c, cls_g)


# speedup vs baseline: 1.0027x; 1.0027x over previous
import jax
import jax.numpy as jnp
from jax import lax
from jax.experimental import pallas as pl
from jax.experimental.pallas import tpu as pltpu
from jax.experimental.pallas import tpu_sc as plsc

NUM_CLASS = 100000
BATCH = 4096
CTX_DIM = 512
N_CLS_CTX = 4
PROMPT_LEN = 4
SEQ = 77
SUFFIX_LEN = SEQ - (2 * PROMPT_LEN + 1 + N_CLS_CTX)

NC, NS = 2, 16
NW = NC * NS
BPW = BATCH // NW
CHUNK = 16
NCHUNK = BPW // CHUNK

K = 64
NSTEP = BATCH // K
NSPLIT = 4
HEAD = 5
ROW_CLS = 5
ROW_MID = 9
ROW_SFX = 13


def _sc_gather_body(label_hbm, cls_hbm, out_hbm, idx_v, buf0, buf1, gsem, ssem):
    wid = lax.axis_index("s") * NC + lax.axis_index("c")
    base = wid * BPW
    pltpu.sync_copy(label_hbm.at[pl.ds(base, BPW)], idx_v)

    bufs = [buf0, buf1]
    g_h = [None] * NCHUNK
    s_h = [None] * NCHUNK

    def gather(c):
        return pltpu.async_copy(
            cls_hbm.at[idx_v.at[pl.ds(c * CHUNK, CHUNK)]], bufs[c % 2], gsem)

    g_h[0] = gather(0)
    for c in range(NCHUNK):
        if c + 1 < NCHUNK:
            if c - 1 >= 0:
                s_h[c - 1].wait()
            g_h[c + 1] = gather(c + 1)
        g_h[c].wait()
        s_h[c] = pltpu.async_copy(
            bufs[c % 2], out_hbm.at[pl.ds(base + c * CHUNK, CHUNK)], ssem)
    s_h[NCHUNK - 2].wait()
    s_h[NCHUNK - 1].wait()


def _sc_gather(label, cls_ctx):
    mesh = plsc.VectorSubcoreMesh(
        core_axis_name="c", subcore_axis_name="s",
        num_cores=NC, num_subcores=NS)
    return pl.kernel(
        _sc_gather_body,
        out_type=jax.ShapeDtypeStruct((BATCH, N_CLS_CTX, CTX_DIM),
                                      jnp.float32),
        mesh=mesh,
        scratch_types=[
            pltpu.VMEM((BPW,), jnp.int32),
            pltpu.VMEM((CHUNK, N_CLS_CTX, CTX_DIM), jnp.float32),
            pltpu.VMEM((CHUNK, N_CLS_CTX, CTX_DIM), jnp.float32),
            pltpu.SemaphoreType.DMA,
            pltpu.SemaphoreType.DMA,
        ],
        name="cls_gather_sc",
    )(label, cls_ctx)


def _tc_fill_body(head_ref, mid_ref, sfx_ref, cls_ref, out_ref):
    out_ref[:, 0:HEAD, :] = jnp.broadcast_to(
        head_ref[...][None], (K, HEAD, CTX_DIM))
    out_ref[:, ROW_CLS:ROW_MID, :] = cls_ref[...]
    out_ref[:, ROW_MID:ROW_SFX, :] = jnp.broadcast_to(
        mid_ref[...][None], (K, PROMPT_LEN, CTX_DIM))
    out_ref[:, ROW_SFX:SEQ, :] = jnp.broadcast_to(
        sfx_ref[...][None], (K, SUFFIX_LEN, CTX_DIM))


def _tc_fill(head_c, mid_c, sfx_c, cls_g):
    return pl.pallas_call(
        _tc_fill_body,
        grid=(NSTEP,),
        in_specs=[
            pl.BlockSpec((HEAD, CTX_DIM), lambda i: (0, 0)),
            pl.BlockSpec((PROMPT_LEN, CTX_DIM), lambda i: (0, 0)),
            pl.BlockSpec((SUFFIX_LEN, CTX_DIM), lambda i: (0, 0)),
            pl.BlockSpec((K, N_CLS_CTX, CTX_DIM), lambda i: (i, 0, 0)),
        ],
        out_specs=pl.BlockSpec((K, SEQ, CTX_DIM), lambda i: (i, 0, 0)),
        out_shape=jax.ShapeDtypeStruct((BATCH, SEQ, CTX_DIM), jnp.float32),
        compiler_params=pltpu.CompilerParams(
            dimension_semantics=("arbitrary",)),
        name="prompt_fill_tc",
    )(head_c, mid_c, sfx_c, cls_g)


def kernel(label, cls_ctx, token_prefix, token_suffix, prefix_prompt,
           suffix_prompt):
    cls_g = _sc_gather(label, cls_ctx)
    head_c = jnp.concatenate(
        [token_prefix.reshape(1, CTX_DIM),
         prefix_prompt.reshape(PROMPT_LEN, CTX_DIM)], axis=0)
    mid_c = suffix_prompt.reshape(PROMPT_LEN, CTX_DIM)
    sfx_c = token_suffix.reshape(SUFFIX_LEN, CTX_DIM)
    return _tc_fill(head_c, mid_c, sfx_c, cls_g)
